# baseline (speedup 1.0000x reference)
import functools

import jax
import jax.numpy as jnp
from jax import lax
from jax.experimental import pallas as pl
from jax.experimental.pallas import tpu as pltpu
from jax.experimental.pallas import tpu_sc as plsc

_V = 1000000
_D = 64
_NW = 32
_FULL_BLOCKS = _V // 128
_TAIL = _V - _FULL_BLOCKS * 128
_BPW = _FULL_BLOCKS // _NW
_EXTRA = _FULL_BLOCKS % _NW


def _transpose_call(wt, tail1d):
    mesh = plsc.VectorSubcoreMesh(core_axis_name="c", subcore_axis_name="s")

    @functools.partial(
        pl.kernel,
        out_type=jax.ShapeDtypeStruct((_V * _D,), jnp.float32),
        mesh=mesh,
        scratch_types=[
            pltpu.VMEM((64, 128), jnp.float32),
            pltpu.VMEM((64, 128), jnp.float32),
            pltpu.VMEM((8192,), jnp.float32),
            pltpu.VMEM((8192,), jnp.float32),
            pltpu.SemaphoreType.DMA,
            pltpu.SemaphoreType.DMA,
            pltpu.SemaphoreType.DMA,
            pltpu.SemaphoreType.DMA,
        ],
        compiler_params=pltpu.CompilerParams(
            use_tc_tiling_on_sc=True, needs_layout_passes=False
        ),
    )
    def tcall(wt_hbm, tail_hbm, inter_hbm, buf0, buf1, tbuf0, tbuf1,
              ss0, ss1, ws0, ws1):
        bufs = (buf0, buf1)
        tbufs = (tbuf0, tbuf1)
        ssems = (ss0, ss1)
        wsems = (ws0, ws1)
        wid = lax.axis_index("s") * 2 + lax.axis_index("c")
        start = wid * _BPW + jnp.minimum(wid, _EXTRA)
        nblk = _BPW + jnp.where(wid < _EXTRA, 1, 0)
        lanes64 = lax.iota(jnp.int32, 16) * 64

        def stage_start(tc, b):
            for tr in range(8):
                pltpu.async_copy(
                    wt_hbm.at[pl.ds(tr * 8, 8), pl.ds(tc * 128, 128)],
                    bufs[b].at[pl.ds(tr * 8, 8)],
                    ssems[b],
                )

        def stage_wait(tc, b):
            for tr in range(8):
                pltpu.make_async_copy(
                    wt_hbm.at[pl.ds(tr * 8, 8), pl.ds(tc * 128, 128)],
                    bufs[b].at[pl.ds(tr * 8, 8)],
                    ssems[b],
                ).wait()

        def shuffle(b):
            for f in range(64):
                for jg in range(8):
                    vec = bufs[b][f, pl.ds(jg * 16, 16)]
                    plsc.store_scatter(
                        tbufs[b], [lanes64 + (jg * 1024 + f)], vec
                    )

        def write_start(tc, b):
            pltpu.async_copy(
                tbufs[b], inter_hbm.at[pl.ds(tc * 8192, 8192)], wsems[b]
            )

        def write_wait(tc, b):
            pltpu.make_async_copy(
                tbufs[b], inter_hbm.at[pl.ds(tc * 8192, 8192)], wsems[b]
            ).wait()

        stage_start(start, 0)

        def body(k, carry):
            tc = start + k

            def one(b):
                stage_wait(tc, b)

                @pl.when(k + 1 < nblk)
                def _():
                    stage_start(tc + 1, 1 - b)

                @pl.when(k >= 2)
                def _():
                    write_wait(tc - 2, b)

                shuffle(b)
                write_start(tc, b)

            b = jnp.remainder(k, 2)

            @pl.when(b == 0)
            def _():
                one(0)

            @pl.when(b == 1)
            def _():
                one(1)

            return carry

        lax.fori_loop(0, nblk, body, 0)

        @pl.when(jnp.remainder(nblk, 2) == 0)
        def _():
            write_wait(start + nblk - 2, 0)
            write_wait(start + nblk - 1, 1)

        @pl.when(jnp.remainder(nblk, 2) == 1)
        def _():
            write_wait(start + nblk - 2, 1)
            write_wait(start + nblk - 1, 0)

        @pl.when(wid == _NW - 1)
        def _():
            pltpu.sync_copy(tail_hbm, tbuf0.at[pl.ds(0, _TAIL * _D)])
            pltpu.sync_copy(
                tbuf0.at[pl.ds(0, _TAIL * _D)],
                inter_hbm.at[pl.ds(_FULL_BLOCKS * 128 * _D, _TAIL * _D)],
            )

    return tcall(wt, tail1d)


def _gather_call(idxf, inter):
    nch, ch = idxf.shape
    cpw = nch // _NW
    mesh = plsc.VectorSubcoreMesh(core_axis_name="c", subcore_axis_name="s")

    @functools.partial(
        pl.kernel,
        out_type=jax.ShapeDtypeStruct((nch * ch * _D,), jnp.float32),
        mesh=mesh,
        scratch_types=[
            pltpu.VMEM((cpw, ch), jnp.int32),
            pltpu.VMEM((ch, _D), jnp.float32),
            pltpu.VMEM((ch, _D), jnp.float32),
            pltpu.VMEM((8192,), jnp.float32),
            pltpu.VMEM((8192,), jnp.float32),
            pltpu.SemaphoreType.DMA,
            pltpu.SemaphoreType.DMA,
            pltpu.SemaphoreType.DMA,
            pltpu.SemaphoreType.DMA,
        ],
        compiler_params=pltpu.CompilerParams(
            use_tc_tiling_on_sc=False, needs_layout_passes=False
        ),
    )
    def gcall(idx_hbm, inter_hbm, out_hbm, idx_v, buf0, buf1, tbuf0, tbuf1,
              gs0, gs1, ws0, ws1):
        bufs = (buf0, buf1)
        tbufs = (tbuf0, tbuf1)
        gsems = (gs0, gs1)
        wsems = (ws0, ws1)
        wid = lax.axis_index("s") * 2 + lax.axis_index("c")
        c0 = wid * cpw
        pltpu.sync_copy(idx_hbm.at[pl.ds(c0, cpw)], idx_v)
        lanes128 = lax.iota(jnp.int32, 16) * 128

        def g_start(k, b):
            pltpu.async_copy(inter_hbm.at[idx_v.at[k]], bufs[b], gsems[b])

        def g_wait(k, b):
            pltpu.make_async_copy(
                inter_hbm.at[idx_v.at[k]], bufs[b], gsems[b]
            ).wait()

        def out_off(k, tr):
            c = c0 + k
            f = c // 128
            tc = c - f * 128
            return f * 1048576 + tr * 131072 + tc * 1024

        def write_start(k, b):
            for tr in range(8):
                pltpu.async_copy(
                    tbufs[b].at[pl.ds(tr * 1024, 1024)],
                    out_hbm.at[pl.ds(out_off(k, tr), 1024)],
                    wsems[b],
                )

        def write_wait(k, b):
            for tr in range(8):
                pltpu.make_async_copy(
                    tbufs[b].at[pl.ds(tr * 1024, 1024)],
                    out_hbm.at[pl.ds(out_off(k, tr), 1024)],
                    wsems[b],
                ).wait()

        def shuffle(b):
            for j in range(128):
                for fg in range(4):
                    vec = bufs[b][j, pl.ds(fg * 16, 16)]
                    plsc.store_scatter(
                        tbufs[b], [lanes128 + (fg * 2048 + j)], vec
                    )

        g_start(0, 0)
        g_start(1, 1)

        def body(i, carry):
            for b in range(2):
                k = 2 * i + b
                g_wait(k, b)

                @pl.when(i >= 1)
                def _():
                    write_wait(k - 2, b)

                shuffle(b)
                write_start(k, b)
                g_start(k + 2, b)
            return carry

        lax.fori_loop(0, cpw // 2 - 1, body, 0)
        for b in range(2):
            k = cpw - 2 + b
            g_wait(k, b)
            write_wait(k - 2, b)
            shuffle(b)
            write_start(k, b)
            write_wait(k, b)

    return gcall(idxf, inter)


@jax.jit
def _embed(x, weight):
    batch, fields = x.shape
    idxf = x.T.reshape(fields * (batch // 128), 128).astype(jnp.int32)
    tail1d = weight[_FULL_BLOCKS * 128 :].reshape(_TAIL * _D)
    inter = _transpose_call(weight.T, tail1d).reshape(_V, _D)
    out1d = _gather_call(idxf, inter)
    out5 = out1d.reshape(fields, 8, batch // 128, 8, 128)
    return out5.transpose(2, 4, 0, 1, 3).reshape(batch, fields, _D)


def kernel(x, weight):
    return _embed(x, weight)


# speedup vs baseline: 2.7074x; 2.7074x over previous
import functools

import jax
import jax.numpy as jnp
from jax import lax
from jax.experimental import pallas as pl
from jax.experimental.pallas import tpu as pltpu
from jax.experimental.pallas import tpu_sc as plsc

_V = 1000000
_D = 64
_NW = 32
_FULL_BLOCKS = _V // 128
_TAIL = _V - _FULL_BLOCKS * 128
_BPW = _FULL_BLOCKS // _NW
_EXTRA = _FULL_BLOCKS % _NW


def _transpose_call(wt, tail1d):
    mesh = plsc.VectorSubcoreMesh(core_axis_name="c", subcore_axis_name="s")

    @functools.partial(
        pl.kernel,
        out_type=jax.ShapeDtypeStruct((_V * _D,), jnp.float32),
        mesh=mesh,
        scratch_types=[
            pltpu.VMEM((64, 128), jnp.float32),
            pltpu.VMEM((64, 128), jnp.float32),
            pltpu.VMEM((8192,), jnp.float32),
            pltpu.VMEM((8192,), jnp.float32),
            pltpu.SemaphoreType.DMA,
            pltpu.SemaphoreType.DMA,
            pltpu.SemaphoreType.DMA,
            pltpu.SemaphoreType.DMA,
        ],
        compiler_params=pltpu.CompilerParams(
            use_tc_tiling_on_sc=True, needs_layout_passes=False
        ),
    )
    def tcall(wt_hbm, tail_hbm, inter_hbm, buf0, buf1, tbuf0, tbuf1,
              ss0, ss1, ws0, ws1):
        bufs = (buf0, buf1)
        tbufs = (tbuf0, tbuf1)
        ssems = (ss0, ss1)
        wsems = (ws0, ws1)
        wid = lax.axis_index("s") * 2 + lax.axis_index("c")
        start = wid * _BPW + jnp.minimum(wid, _EXTRA)
        nblk = _BPW + jnp.where(wid < _EXTRA, 1, 0)
        lanes = lax.iota(jnp.int32, 16)
        lanes64 = lanes * 64

        def stage_start(tc, b):
            for tr in range(8):
                pltpu.async_copy(
                    wt_hbm.at[pl.ds(tr * 8, 8), pl.ds(tc * 128, 128)],
                    bufs[b].at[pl.ds(tr * 8, 8)],
                    ssems[b],
                )

        def stage_wait(tc, b):
            for tr in range(8):
                pltpu.make_async_copy(
                    wt_hbm.at[pl.ds(tr * 8, 8), pl.ds(tc * 128, 128)],
                    bufs[b].at[pl.ds(tr * 8, 8)],
                    ssems[b],
                ).wait()

        def shuffle(b):
            def sbody(s, carry):
                rot = jnp.bitwise_and(lanes + s, 15)
                dbase = lanes64 + rot
                for f0 in range(0, 64, 16):
                    for j0 in range(0, 128, 16):
                        vec = plsc.load_gather(
                            bufs[b], [f0 + rot, j0 + lanes]
                        )
                        plsc.store_scatter(
                            tbufs[b], [dbase + (j0 * 64 + f0)], vec
                        )
                return carry

            lax.fori_loop(0, 16, sbody, 0)

        def write_start(tc, b):
            pltpu.async_copy(
                tbufs[b], inter_hbm.at[pl.ds(tc * 8192, 8192)], wsems[b]
            )

        def write_wait(tc, b):
            pltpu.make_async_copy(
                tbufs[b], inter_hbm.at[pl.ds(tc * 8192, 8192)], wsems[b]
            ).wait()

        stage_start(start, 0)

        def body(k, carry):
            tc = start + k

            def one(b):
                stage_wait(tc, b)

                @pl.when(k + 1 < nblk)
                def _():
                    stage_start(tc + 1, 1 - b)

                @pl.when(k >= 2)
                def _():
                    write_wait(tc - 2, b)

                shuffle(b)
                write_start(tc, b)

            b = jnp.remainder(k, 2)

            @pl.when(b == 0)
            def _():
                one(0)

            @pl.when(b == 1)
            def _():
                one(1)

            return carry

        lax.fori_loop(0, nblk, body, 0)

        @pl.when(jnp.remainder(nblk, 2) == 0)
        def _():
            write_wait(start + nblk - 2, 0)
            write_wait(start + nblk - 1, 1)

        @pl.when(jnp.remainder(nblk, 2) == 1)
        def _():
            write_wait(start + nblk - 2, 1)
            write_wait(start + nblk - 1, 0)

        @pl.when(wid == _NW - 1)
        def _():
            pltpu.sync_copy(tail_hbm, tbuf0.at[pl.ds(0, _TAIL * _D)])
            pltpu.sync_copy(
                tbuf0.at[pl.ds(0, _TAIL * _D)],
                inter_hbm.at[pl.ds(_FULL_BLOCKS * 128 * _D, _TAIL * _D)],
            )

    return tcall(wt, tail1d)


def _gather_call(idxf, inter):
    nch, ch = idxf.shape
    cpw = nch // _NW
    mesh = plsc.VectorSubcoreMesh(core_axis_name="c", subcore_axis_name="s")

    @functools.partial(
        pl.kernel,
        out_type=jax.ShapeDtypeStruct((nch * ch * _D,), jnp.float32),
        mesh=mesh,
        scratch_types=[
            pltpu.VMEM((cpw, ch), jnp.int32),
            pltpu.VMEM((ch, _D), jnp.float32),
            pltpu.VMEM((ch, _D), jnp.float32),
            pltpu.VMEM((8192,), jnp.float32),
            pltpu.VMEM((8192,), jnp.float32),
            pltpu.SemaphoreType.DMA,
            pltpu.SemaphoreType.DMA,
            pltpu.SemaphoreType.DMA,
            pltpu.SemaphoreType.DMA,
        ],
        compiler_params=pltpu.CompilerParams(
            use_tc_tiling_on_sc=False, needs_layout_passes=False
        ),
    )
    def gcall(idx_hbm, inter_hbm, out_hbm, idx_v, buf0, buf1, tbuf0, tbuf1,
              gs0, gs1, ws0, ws1):
        bufs = (buf0, buf1)
        tbufs = (tbuf0, tbuf1)
        gsems = (gs0, gs1)
        wsems = (ws0, ws1)
        wid = lax.axis_index("s") * 2 + lax.axis_index("c")
        c0 = wid * cpw
        pltpu.sync_copy(idx_hbm.at[pl.ds(c0, cpw)], idx_v)
        lanes = lax.iota(jnp.int32, 16)
        lanes128 = lanes * 128

        def g_start(k, b):
            pltpu.async_copy(inter_hbm.at[idx_v.at[k]], bufs[b], gsems[b])

        def g_wait(k, b):
            pltpu.make_async_copy(
                inter_hbm.at[idx_v.at[k]], bufs[b], gsems[b]
            ).wait()

        def out_off(k, tr):
            c = c0 + k
            f = c // 128
            tc = c - f * 128
            return f * 1048576 + tr * 131072 + tc * 1024

        def write_start(k, b):
            for tr in range(8):
                pltpu.async_copy(
                    tbufs[b].at[pl.ds(tr * 1024, 1024)],
                    out_hbm.at[pl.ds(out_off(k, tr), 1024)],
                    wsems[b],
                )

        def write_wait(k, b):
            for tr in range(8):
                pltpu.make_async_copy(
                    tbufs[b].at[pl.ds(tr * 1024, 1024)],
                    out_hbm.at[pl.ds(out_off(k, tr), 1024)],
                    wsems[b],
                ).wait()

        def shuffle(b):
            def sbody(s, carry):
                rot = jnp.bitwise_and(lanes + s, 15)
                dbase = lanes128 + rot
                for j0 in range(0, 128, 16):
                    for d0 in range(0, 64, 16):
                        vec = plsc.load_gather(
                            bufs[b], [j0 + rot, d0 + lanes]
                        )
                        plsc.store_scatter(
                            tbufs[b], [dbase + (d0 * 128 + j0)], vec
                        )
                return carry

            lax.fori_loop(0, 16, sbody, 0)

        g_start(0, 0)
        g_start(1, 1)

        def body(i, carry):
            for b in range(2):
                k = 2 * i + b
                g_wait(k, b)

                @pl.when(i >= 1)
                def _():
                    write_wait(k - 2, b)

                shuffle(b)
                write_start(k, b)
                g_start(k + 2, b)
            return carry

        lax.fori_loop(0, cpw // 2 - 1, body, 0)
        for b in range(2):
            k = cpw - 2 + b
            g_wait(k, b)
            write_wait(k - 2, b)
            shuffle(b)
            write_start(k, b)
            write_wait(k, b)

    return gcall(idxf, inter)


@jax.jit
def _embed(x, weight):
    batch, fields = x.shape
    idxf = x.T.reshape(fields * (batch // 128), 128).astype(jnp.int32)
    tail1d = weight[_FULL_BLOCKS * 128 :].reshape(_TAIL * _D)
    inter = _transpose_call(weight.T, tail1d).reshape(_V, _D)
    out1d = _gather_call(idxf, inter)
    out5 = out1d.reshape(fields, 8, batch // 128, 8, 128)
    return out5.transpose(2, 4, 0, 1, 3).reshape(batch, fields, _D)


def kernel(x, weight):
    return _embed(x, weight)


# speedup vs baseline: 4.1057x; 1.5165x over previous
import functools

import jax
import jax.numpy as jnp
from jax import lax
from jax.experimental import pallas as pl
from jax.experimental.pallas import tpu as pltpu
from jax.experimental.pallas import tpu_sc as plsc

_V = 1000000
_D = 64
_NW = 32
_FULL_BLOCKS = _V // 128
_TAIL = _V - _FULL_BLOCKS * 128
_BPW = _FULL_BLOCKS // _NW
_EXTRA = _FULL_BLOCKS % _NW


def _transpose_call(wt, tail1d):
    mesh = plsc.VectorSubcoreMesh(core_axis_name="c", subcore_axis_name="s")

    @functools.partial(
        pl.kernel,
        out_type=jax.ShapeDtypeStruct((_V * _D,), jnp.float32),
        mesh=mesh,
        scratch_types=[
            pltpu.VMEM((64, 128), jnp.float32),
            pltpu.VMEM((64, 128), jnp.float32),
            pltpu.VMEM((8192,), jnp.float32),
            pltpu.VMEM((8192,), jnp.float32),
            pltpu.SemaphoreType.DMA,
            pltpu.SemaphoreType.DMA,
            pltpu.SemaphoreType.DMA,
            pltpu.SemaphoreType.DMA,
        ],
        compiler_params=pltpu.CompilerParams(
            use_tc_tiling_on_sc=True, needs_layout_passes=False
        ),
    )
    def tcall(wt_hbm, tail_hbm, inter_hbm, buf0, buf1, tbuf0, tbuf1,
              ss0, ss1, ws0, ws1):
        bufs = (buf0, buf1)
        tbufs = (tbuf0, tbuf1)
        ssems = (ss0, ss1)
        wsems = (ws0, ws1)
        wid = lax.axis_index("s") * 2 + lax.axis_index("c")
        start = wid * _BPW + jnp.minimum(wid, _EXTRA)
        nblk = _BPW + jnp.where(wid < _EXTRA, 1, 0)
        lanes = lax.iota(jnp.int32, 16)
        lanes64 = lanes * 64

        def stage_start(tc, b):
            for tr in range(8):
                pltpu.async_copy(
                    wt_hbm.at[pl.ds(tr * 8, 8), pl.ds(tc * 128, 128)],
                    bufs[b].at[pl.ds(tr * 8, 8)],
                    ssems[b],
                )

        def stage_wait(tc, b):
            for tr in range(8):
                pltpu.make_async_copy(
                    wt_hbm.at[pl.ds(tr * 8, 8), pl.ds(tc * 128, 128)],
                    bufs[b].at[pl.ds(tr * 8, 8)],
                    ssems[b],
                ).wait()

        def shuffle(b):
            @plsc.parallel_loop(0, 16, unroll=4)
            def sbody(s):
                rot = jnp.bitwise_and(lanes + s, 15)
                dbase = lanes64 + rot
                for f0 in range(0, 64, 16):
                    for j0 in range(0, 128, 16):
                        vec = plsc.load_gather(
                            bufs[b], [f0 + rot, j0 + lanes]
                        )
                        plsc.store_scatter(
                            tbufs[b], [dbase + (j0 * 64 + f0)], vec
                        )

        def write_start(tc, b):
            pltpu.async_copy(
                tbufs[b], inter_hbm.at[pl.ds(tc * 8192, 8192)], wsems[b]
            )

        def write_wait(tc, b):
            pltpu.make_async_copy(
                tbufs[b], inter_hbm.at[pl.ds(tc * 8192, 8192)], wsems[b]
            ).wait()

        stage_start(start, 0)

        def body(k, carry):
            tc = start + k

            def one(b):
                stage_wait(tc, b)

                @pl.when(k + 1 < nblk)
                def _():
                    stage_start(tc + 1, 1 - b)

                @pl.when(k >= 2)
                def _():
                    write_wait(tc - 2, b)

                shuffle(b)
                write_start(tc, b)

            b = jnp.remainder(k, 2)

            @pl.when(b == 0)
            def _():
                one(0)

            @pl.when(b == 1)
            def _():
                one(1)

            return carry

        lax.fori_loop(0, nblk, body, 0)

        @pl.when(jnp.remainder(nblk, 2) == 0)
        def _():
            write_wait(start + nblk - 2, 0)
            write_wait(start + nblk - 1, 1)

        @pl.when(jnp.remainder(nblk, 2) == 1)
        def _():
            write_wait(start + nblk - 2, 1)
            write_wait(start + nblk - 1, 0)

        @pl.when(wid == _NW - 1)
        def _():
            pltpu.sync_copy(tail_hbm, tbuf0.at[pl.ds(0, _TAIL * _D)])
            pltpu.sync_copy(
                tbuf0.at[pl.ds(0, _TAIL * _D)],
                inter_hbm.at[pl.ds(_FULL_BLOCKS * 128 * _D, _TAIL * _D)],
            )

    return tcall(wt, tail1d)


def _gather_call(idxf, inter):
    nch, ch = idxf.shape
    cpw = nch // _NW
    mesh = plsc.VectorSubcoreMesh(core_axis_name="c", subcore_axis_name="s")

    @functools.partial(
        pl.kernel,
        out_type=jax.ShapeDtypeStruct((nch * ch * _D,), jnp.float32),
        mesh=mesh,
        scratch_types=[
            pltpu.VMEM((cpw, ch), jnp.int32),
            pltpu.VMEM((ch, _D), jnp.float32),
            pltpu.VMEM((ch, _D), jnp.float32),
            pltpu.VMEM((8192,), jnp.float32),
            pltpu.VMEM((8192,), jnp.float32),
            pltpu.SemaphoreType.DMA,
            pltpu.SemaphoreType.DMA,
            pltpu.SemaphoreType.DMA,
            pltpu.SemaphoreType.DMA,
        ],
        compiler_params=pltpu.CompilerParams(
            use_tc_tiling_on_sc=False, needs_layout_passes=False
        ),
    )
    def gcall(idx_hbm, inter_hbm, out_hbm, idx_v, buf0, buf1, tbuf0, tbuf1,
              gs0, gs1, ws0, ws1):
        bufs = (buf0, buf1)
        tbufs = (tbuf0, tbuf1)
        gsems = (gs0, gs1)
        wsems = (ws0, ws1)
        wid = lax.axis_index("s") * 2 + lax.axis_index("c")
        c0 = wid * cpw
        pltpu.sync_copy(idx_hbm.at[pl.ds(c0, cpw)], idx_v)
        lanes = lax.iota(jnp.int32, 16)
        lanes128 = lanes * 128

        def g_start(k, b):
            pltpu.async_copy(inter_hbm.at[idx_v.at[k]], bufs[b], gsems[b])

        def g_wait(k, b):
            pltpu.make_async_copy(
                inter_hbm.at[idx_v.at[k]], bufs[b], gsems[b]
            ).wait()

        def out_off(k, tr):
            c = c0 + k
            f = c // 128
            tc = c - f * 128
            return f * 1048576 + tr * 131072 + tc * 1024

        def write_start(k, b):
            for tr in range(8):
                pltpu.async_copy(
                    tbufs[b].at[pl.ds(tr * 1024, 1024)],
                    out_hbm.at[pl.ds(out_off(k, tr), 1024)],
                    wsems[b],
                )

        def write_wait(k, b):
            for tr in range(8):
                pltpu.make_async_copy(
                    tbufs[b].at[pl.ds(tr * 1024, 1024)],
                    out_hbm.at[pl.ds(out_off(k, tr), 1024)],
                    wsems[b],
                ).wait()

        def shuffle(b):
            @plsc.parallel_loop(0, 16, unroll=4)
            def sbody(s):
                rot = jnp.bitwise_and(lanes + s, 15)
                dbase = lanes128 + rot
                for j0 in range(0, 128, 16):
                    for d0 in range(0, 64, 16):
                        vec = plsc.load_gather(
                            bufs[b], [j0 + rot, d0 + lanes]
                        )
                        plsc.store_scatter(
                            tbufs[b], [dbase + (d0 * 128 + j0)], vec
                        )

        g_start(0, 0)
        g_start(1, 1)

        def body(i, carry):
            for b in range(2):
                k = 2 * i + b
                g_wait(k, b)

                @pl.when(i >= 1)
                def _():
                    write_wait(k - 2, b)

                shuffle(b)
                write_start(k, b)
                g_start(k + 2, b)
            return carry

        lax.fori_loop(0, cpw // 2 - 1, body, 0)
        for b in range(2):
            k = cpw - 2 + b
            g_wait(k, b)
            write_wait(k - 2, b)
            shuffle(b)
            write_start(k, b)
            write_wait(k, b)

    return gcall(idxf, inter)


@jax.jit
def _embed(x, weight):
    batch, fields = x.shape
    idxf = x.T.reshape(fields * (batch // 128), 128).astype(jnp.int32)
    tail1d = weight[_FULL_BLOCKS * 128 :].reshape(_TAIL * _D)
    inter = _transpose_call(weight.T, tail1d).reshape(_V, _D)
    out1d = _gather_call(idxf, inter)
    out5 = out1d.reshape(fields, 8, batch // 128, 8, 128)
    return out5.transpose(2, 4, 0, 1, 3).reshape(batch, fields, _D)


def kernel(x, weight):
    return _embed(x, weight)


# speedup vs baseline: 4.5475x; 1.1076x over previous
import functools

import jax
import jax.numpy as jnp
from jax import lax
from jax.experimental import pallas as pl
from jax.experimental.pallas import tpu as pltpu
from jax.experimental.pallas import tpu_sc as plsc

_V = 1000000
_D = 64
_NW = 32
_FULL_BLOCKS = _V // 128
_TAIL = _V - _FULL_BLOCKS * 128
_BPW = _FULL_BLOCKS // _NW
_EXTRA = _FULL_BLOCKS % _NW


def _transpose_call(wt, tail1d):
    mesh = plsc.VectorSubcoreMesh(core_axis_name="c", subcore_axis_name="s")

    @functools.partial(
        pl.kernel,
        out_type=jax.ShapeDtypeStruct((_V * _D,), jnp.float32),
        mesh=mesh,
        scratch_types=[
            pltpu.VMEM((64, 128), jnp.float32),
            pltpu.VMEM((64, 128), jnp.float32),
            pltpu.VMEM((8192,), jnp.float32),
            pltpu.VMEM((8192,), jnp.float32),
            pltpu.SemaphoreType.DMA,
            pltpu.SemaphoreType.DMA,
            pltpu.SemaphoreType.DMA,
            pltpu.SemaphoreType.DMA,
        ],
        compiler_params=pltpu.CompilerParams(
            use_tc_tiling_on_sc=True, needs_layout_passes=False
        ),
    )
    def tcall(wt_hbm, tail_hbm, inter_hbm, buf0, buf1, tbuf0, tbuf1,
              ss0, ss1, ws0, ws1):
        bufs = (buf0, buf1)
        tbufs = (tbuf0, tbuf1)
        ssems = (ss0, ss1)
        wsems = (ws0, ws1)
        wid = lax.axis_index("s") * 2 + lax.axis_index("c")
        start = wid * _BPW + jnp.minimum(wid, _EXTRA)
        nblk = _BPW + jnp.where(wid < _EXTRA, 1, 0)
        lanes = lax.iota(jnp.int32, 16)
        lanes64 = lanes * 64

        def stage_start(tc, b):
            for tr in range(8):
                pltpu.async_copy(
                    wt_hbm.at[pl.ds(tr * 8, 8), pl.ds(tc * 128, 128)],
                    bufs[b].at[pl.ds(tr * 8, 8)],
                    ssems[b],
                )

        def stage_wait(tc, b):
            for tr in range(8):
                pltpu.make_async_copy(
                    wt_hbm.at[pl.ds(tr * 8, 8), pl.ds(tc * 128, 128)],
                    bufs[b].at[pl.ds(tr * 8, 8)],
                    ssems[b],
                ).wait()

        def shuffle(b):
            @plsc.parallel_loop(0, 16, unroll=8)
            def sbody(s):
                rot = jnp.bitwise_and(lanes + s, 15)
                dbase = lanes64 + rot
                for f0 in range(0, 64, 16):
                    for j0 in range(0, 128, 16):
                        vec = plsc.load_gather(
                            bufs[b], [f0 + rot, j0 + lanes]
                        )
                        plsc.store_scatter(
                            tbufs[b], [dbase + (j0 * 64 + f0)], vec
                        )

        def write_start(tc, b):
            pltpu.async_copy(
                tbufs[b], inter_hbm.at[pl.ds(tc * 8192, 8192)], wsems[b]
            )

        def write_wait(tc, b):
            pltpu.make_async_copy(
                tbufs[b], inter_hbm.at[pl.ds(tc * 8192, 8192)], wsems[b]
            ).wait()

        stage_start(start, 0)

        def body(k, carry):
            tc = start + k

            def one(b):
                stage_wait(tc, b)

                @pl.when(k + 1 < nblk)
                def _():
                    stage_start(tc + 1, 1 - b)

                @pl.when(k >= 2)
                def _():
                    write_wait(tc - 2, b)

                shuffle(b)
                write_start(tc, b)

            b = jnp.remainder(k, 2)

            @pl.when(b == 0)
            def _():
                one(0)

            @pl.when(b == 1)
            def _():
                one(1)

            return carry

        lax.fori_loop(0, nblk, body, 0)

        @pl.when(jnp.remainder(nblk, 2) == 0)
        def _():
            write_wait(start + nblk - 2, 0)
            write_wait(start + nblk - 1, 1)

        @pl.when(jnp.remainder(nblk, 2) == 1)
        def _():
            write_wait(start + nblk - 2, 1)
            write_wait(start + nblk - 1, 0)

        @pl.when(wid == _NW - 1)
        def _():
            pltpu.sync_copy(tail_hbm, tbuf0.at[pl.ds(0, _TAIL * _D)])
            pltpu.sync_copy(
                tbuf0.at[pl.ds(0, _TAIL * _D)],
                inter_hbm.at[pl.ds(_FULL_BLOCKS * 128 * _D, _TAIL * _D)],
            )

    return tcall(wt, tail1d)


def _gather_call(idxf, inter):
    nch, ch = idxf.shape
    cpw = nch // _NW
    mesh = plsc.VectorSubcoreMesh(core_axis_name="c", subcore_axis_name="s")

    @functools.partial(
        pl.kernel,
        out_type=jax.ShapeDtypeStruct((nch * ch * _D,), jnp.float32),
        mesh=mesh,
        scratch_types=[
            pltpu.VMEM((cpw, ch), jnp.int32),
            pltpu.VMEM((ch, _D), jnp.float32),
            pltpu.VMEM((ch, _D), jnp.float32),
            pltpu.VMEM((8192,), jnp.float32),
            pltpu.VMEM((8192,), jnp.float32),
            pltpu.SemaphoreType.DMA,
            pltpu.SemaphoreType.DMA,
            pltpu.SemaphoreType.DMA,
            pltpu.SemaphoreType.DMA,
        ],
        compiler_params=pltpu.CompilerParams(
            use_tc_tiling_on_sc=False, needs_layout_passes=False
        ),
    )
    def gcall(idx_hbm, inter_hbm, out_hbm, idx_v, buf0, buf1, tbuf0, tbuf1,
              gs0, gs1, ws0, ws1):
        bufs = (buf0, buf1)
        tbufs = (tbuf0, tbuf1)
        gsems = (gs0, gs1)
        wsems = (ws0, ws1)
        wid = lax.axis_index("s") * 2 + lax.axis_index("c")
        c0 = wid * cpw
        pltpu.sync_copy(idx_hbm.at[pl.ds(c0, cpw)], idx_v)
        lanes = lax.iota(jnp.int32, 16)
        lanes128 = lanes * 128

        def g_start(k, b):
            pltpu.async_copy(inter_hbm.at[idx_v.at[k]], bufs[b], gsems[b])

        def g_wait(k, b):
            pltpu.make_async_copy(
                inter_hbm.at[idx_v.at[k]], bufs[b], gsems[b]
            ).wait()

        def out_off(k, tr):
            c = c0 + k
            f = c // 128
            tc = c - f * 128
            return f * 1048576 + tr * 131072 + tc * 1024

        def write_start(k, b):
            for tr in range(8):
                pltpu.async_copy(
                    tbufs[b].at[pl.ds(tr * 1024, 1024)],
                    out_hbm.at[pl.ds(out_off(k, tr), 1024)],
                    wsems[b],
                )

        def write_wait(k, b):
            for tr in range(8):
                pltpu.make_async_copy(
                    tbufs[b].at[pl.ds(tr * 1024, 1024)],
                    out_hbm.at[pl.ds(out_off(k, tr), 1024)],
                    wsems[b],
                ).wait()

        def shuffle(b):
            @plsc.parallel_loop(0, 16, unroll=8)
            def sbody(s):
                rot = jnp.bitwise_and(lanes + s, 15)
                dbase = lanes128 + rot
                for j0 in range(0, 128, 16):
                    for d0 in range(0, 64, 16):
                        vec = plsc.load_gather(
                            bufs[b], [j0 + rot, d0 + lanes]
                        )
                        plsc.store_scatter(
                            tbufs[b], [dbase + (d0 * 128 + j0)], vec
                        )

        g_start(0, 0)
        g_start(1, 1)

        def body(i, carry):
            for b in range(2):
                k = 2 * i + b
                g_wait(k, b)

                @pl.when(i >= 1)
                def _():
                    write_wait(k - 2, b)

                shuffle(b)
                write_start(k, b)
                g_start(k + 2, b)
            return carry

        lax.fori_loop(0, cpw // 2 - 1, body, 0)
        for b in range(2):
            k = cpw - 2 + b
            g_wait(k, b)
            write_wait(k - 2, b)
            shuffle(b)
            write_start(k, b)
            write_wait(k, b)

    return gcall(idxf, inter)


@jax.jit
def _embed(x, weight):
    batch, fields = x.shape
    idxf = x.T.reshape(fields * (batch // 128), 128).astype(jnp.int32)
    tail1d = weight[_FULL_BLOCKS * 128 :].reshape(_TAIL * _D)
    inter = _transpose_call(weight.T, tail1d).reshape(_V, _D)
    out1d = _gather_call(idxf, inter)
    out5 = out1d.reshape(fields, 8, batch // 128, 8, 128)
    return out5.transpose(2, 4, 0, 1, 3).reshape(batch, fields, _D)


def kernel(x, weight):
    return _embed(x, weight)


# speedup vs baseline: 5.6615x; 1.2450x over previous
import functools

import jax
import jax.numpy as jnp
from jax import lax
from jax.experimental import pallas as pl
from jax.experimental.pallas import tpu as pltpu
from jax.experimental.pallas import tpu_sc as plsc

_V = 1000000
_D = 64
_NW = 32
_FULL_BLOCKS = _V // 128
_TAIL = _V - _FULL_BLOCKS * 128
_BPW = _FULL_BLOCKS // _NW
_EXTRA = _FULL_BLOCKS % _NW


def _transpose_call(wt, tail1d):
    mesh = plsc.VectorSubcoreMesh(core_axis_name="c", subcore_axis_name="s")

    @functools.partial(
        pl.kernel,
        out_type=jax.ShapeDtypeStruct((_V * _D,), jnp.float32),
        mesh=mesh,
        scratch_types=[
            pltpu.VMEM((64, 256), jnp.float32),
            pltpu.VMEM((64, 256), jnp.float32),
            pltpu.VMEM((16384,), jnp.float32),
            pltpu.VMEM((16384,), jnp.float32),
            pltpu.SemaphoreType.DMA,
            pltpu.SemaphoreType.DMA,
            pltpu.SemaphoreType.DMA,
            pltpu.SemaphoreType.DMA,
        ],
        compiler_params=pltpu.CompilerParams(
            use_tc_tiling_on_sc=True, needs_layout_passes=False
        ),
    )
    def tcall(wt_hbm, tail_hbm, inter_hbm, buf0, buf1, tbuf0, tbuf1,
              ss0, ss1, ws0, ws1):
        bufs = (buf0, buf1)
        tbufs = (tbuf0, tbuf1)
        ssems = (ss0, ss1)
        wsems = (ws0, ws1)
        wid = lax.axis_index("s") * 2 + lax.axis_index("c")
        start = wid * _BPW + 2 * jnp.minimum(wid, _EXTRA // 2)
        rounds = (_BPW + 2 * jnp.where(wid < _EXTRA // 2, 1, 0)) // 2
        lanes = lax.iota(jnp.int32, 16)
        lanes64 = lanes * 64

        def stage_start(tc, b):
            for tr in range(8):
                pltpu.async_copy(
                    wt_hbm.at[pl.ds(tr * 8, 8), pl.ds(tc * 128, 256)],
                    bufs[b].at[pl.ds(tr * 8, 8)],
                    ssems[b],
                )

        def stage_wait(tc, b):
            for tr in range(8):
                pltpu.make_async_copy(
                    wt_hbm.at[pl.ds(tr * 8, 8), pl.ds(tc * 128, 256)],
                    bufs[b].at[pl.ds(tr * 8, 8)],
                    ssems[b],
                ).wait()

        def shuffle(b):
            @plsc.parallel_loop(0, 16, unroll=4)
            def sbody(s):
                rot = jnp.bitwise_and(lanes + s, 15)
                dbase = lanes64 + rot
                for f0 in range(0, 64, 16):
                    for j0 in range(0, 256, 16):
                        vec = plsc.load_gather(
                            bufs[b], [f0 + rot, j0 + lanes]
                        )
                        plsc.store_scatter(
                            tbufs[b], [dbase + (j0 * 64 + f0)], vec
                        )

        def write_start(tc, b):
            pltpu.async_copy(
                tbufs[b], inter_hbm.at[pl.ds(tc * 8192, 16384)], wsems[b]
            )

        def write_wait(tc, b):
            pltpu.make_async_copy(
                tbufs[b], inter_hbm.at[pl.ds(tc * 8192, 16384)], wsems[b]
            ).wait()

        stage_start(start, 0)

        def body(r, carry):
            tc = start + 2 * r

            def one(b):
                stage_wait(tc, b)

                @pl.when(r + 1 < rounds)
                def _():
                    stage_start(tc + 2, 1 - b)

                @pl.when(r >= 2)
                def _():
                    write_wait(tc - 4, b)

                shuffle(b)
                write_start(tc, b)

            b = jnp.remainder(r, 2)

            @pl.when(b == 0)
            def _():
                one(0)

            @pl.when(b == 1)
            def _():
                one(1)

            return carry

        lax.fori_loop(0, rounds, body, 0)

        @pl.when(jnp.remainder(rounds, 2) == 0)
        def _():
            write_wait(start + 2 * rounds - 4, 0)
            write_wait(start + 2 * rounds - 2, 1)

        @pl.when(jnp.remainder(rounds, 2) == 1)
        def _():
            write_wait(start + 2 * rounds - 4, 1)
            write_wait(start + 2 * rounds - 2, 0)

        @pl.when(wid == _NW - 1)
        def _():
            pltpu.sync_copy(tail_hbm, tbuf0.at[pl.ds(0, _TAIL * _D)])
            pltpu.sync_copy(
                tbuf0.at[pl.ds(0, _TAIL * _D)],
                inter_hbm.at[pl.ds(_FULL_BLOCKS * 128 * _D, _TAIL * _D)],
            )

    return tcall(wt, tail1d)


def _gather_call(idxf, inter):
    nch, ch = idxf.shape
    cpw = nch // _NW
    mesh = plsc.VectorSubcoreMesh(core_axis_name="c", subcore_axis_name="s")

    @functools.partial(
        pl.kernel,
        out_type=jax.ShapeDtypeStruct((nch * ch * _D,), jnp.float32),
        mesh=mesh,
        scratch_types=[
            pltpu.VMEM((cpw, ch), jnp.int32),
            pltpu.VMEM((ch, _D), jnp.float32),
            pltpu.VMEM((ch, _D), jnp.float32),
            pltpu.VMEM((8192,), jnp.float32),
            pltpu.VMEM((8192,), jnp.float32),
            pltpu.SemaphoreType.DMA,
            pltpu.SemaphoreType.DMA,
            pltpu.SemaphoreType.DMA,
            pltpu.SemaphoreType.DMA,
        ],
        compiler_params=pltpu.CompilerParams(
            use_tc_tiling_on_sc=False, needs_layout_passes=False
        ),
    )
    def gcall(idx_hbm, inter_hbm, out_hbm, idx_v, buf0, buf1, tbuf0, tbuf1,
              gs0, gs1, ws0, ws1):
        bufs = (buf0, buf1)
        tbufs = (tbuf0, tbuf1)
        gsems = (gs0, gs1)
        wsems = (ws0, ws1)
        wid = lax.axis_index("s") * 2 + lax.axis_index("c")
        c0 = wid * cpw
        pltpu.sync_copy(idx_hbm.at[pl.ds(c0, cpw)], idx_v)
        lanes = lax.iota(jnp.int32, 16)
        lanes128 = lanes * 128

        def g_start(k, b):
            pltpu.async_copy(inter_hbm.at[idx_v.at[k]], bufs[b], gsems[b])

        def g_wait(k, b):
            pltpu.make_async_copy(
                inter_hbm.at[idx_v.at[k]], bufs[b], gsems[b]
            ).wait()

        def out_off(k, tr):
            c = c0 + k
            f = c // 128
            tc = c - f * 128
            return f * 1048576 + tr * 131072 + tc * 1024

        def write_start(k, b):
            for tr in range(8):
                pltpu.async_copy(
                    tbufs[b].at[pl.ds(tr * 1024, 1024)],
                    out_hbm.at[pl.ds(out_off(k, tr), 1024)],
                    wsems[b],
                )

        def write_wait(k, b):
            for tr in range(8):
                pltpu.make_async_copy(
                    tbufs[b].at[pl.ds(tr * 1024, 1024)],
                    out_hbm.at[pl.ds(out_off(k, tr), 1024)],
                    wsems[b],
                ).wait()

        def shuffle(b):
            @plsc.parallel_loop(0, 16, unroll=8)
            def sbody(s):
                rot = jnp.bitwise_and(lanes + s, 15)
                dbase = lanes128 + rot
                for j0 in range(0, 128, 16):
                    for d0 in range(0, 64, 16):
                        vec = plsc.load_gather(
                            bufs[b], [j0 + rot, d0 + lanes]
                        )
                        plsc.store_scatter(
                            tbufs[b], [dbase + (d0 * 128 + j0)], vec
                        )

        g_start(0, 0)
        g_start(1, 1)

        def body(i, carry):
            for b in range(2):
                k = 2 * i + b
                g_wait(k, b)

                @pl.when(i >= 1)
                def _():
                    write_wait(k - 2, b)

                shuffle(b)
                write_start(k, b)
                g_start(k + 2, b)
            return carry

        lax.fori_loop(0, cpw // 2 - 1, body, 0)
        for b in range(2):
            k = cpw - 2 + b
            g_wait(k, b)
            write_wait(k - 2, b)
            shuffle(b)
            write_start(k, b)
            write_wait(k, b)

    return gcall(idxf, inter)


@jax.jit
def _embed(x, weight):
    batch, fields = x.shape
    idxf = x.T.reshape(fields * (batch // 128), 128).astype(jnp.int32)
    tail1d = weight[_FULL_BLOCKS * 128 :].reshape(_TAIL * _D)
    inter = _transpose_call(weight.T, tail1d).reshape(_V, _D)
    out1d = _gather_call(idxf, inter)
    out5 = out1d.reshape(fields, 8, batch // 128, 8, 128)
    return out5.transpose(2, 4, 0, 1, 3).reshape(batch, fields, _D)


def kernel(x, weight):
    return _embed(x, weight)


# speedup vs baseline: 6.0281x; 1.0648x over previous
import functools

import jax
import jax.numpy as jnp
from jax import lax
from jax.experimental import pallas as pl
from jax.experimental.pallas import tpu as pltpu
from jax.experimental.pallas import tpu_sc as plsc

_V = 1000000
_D = 64
_NW = 32
_FULL_BLOCKS = _V // 128
_TAIL = _V - _FULL_BLOCKS * 128
_BPW = _FULL_BLOCKS // _NW
_EXTRA = _FULL_BLOCKS % _NW


def _transpose_call(wt, tail1d):
    mesh = plsc.VectorSubcoreMesh(core_axis_name="c", subcore_axis_name="s")

    @functools.partial(
        pl.kernel,
        out_type=jax.ShapeDtypeStruct((_V * _D,), jnp.float32),
        mesh=mesh,
        scratch_types=[
            pltpu.VMEM((64, 256), jnp.float32),
            pltpu.VMEM((64, 256), jnp.float32),
            pltpu.VMEM((16384,), jnp.float32),
            pltpu.VMEM((16384,), jnp.float32),
            pltpu.SemaphoreType.DMA,
            pltpu.SemaphoreType.DMA,
            pltpu.SemaphoreType.DMA,
            pltpu.SemaphoreType.DMA,
        ],
        compiler_params=pltpu.CompilerParams(
            use_tc_tiling_on_sc=True, needs_layout_passes=False
        ),
    )
    def tcall(wt_hbm, tail_hbm, inter_hbm, buf0, buf1, tbuf0, tbuf1,
              ss0, ss1, ws0, ws1):
        bufs = (buf0, buf1)
        tbufs = (tbuf0, tbuf1)
        ssems = (ss0, ss1)
        wsems = (ws0, ws1)
        wid = lax.axis_index("s") * 2 + lax.axis_index("c")
        start = wid * _BPW + 2 * jnp.minimum(wid, _EXTRA // 2)
        rounds = (_BPW + 2 * jnp.where(wid < _EXTRA // 2, 1, 0)) // 2
        lanes = lax.iota(jnp.int32, 16)
        lanes64 = lanes * 64

        def stage_start(tc, b):
            for tr in range(8):
                pltpu.async_copy(
                    wt_hbm.at[pl.ds(tr * 8, 8), pl.ds(tc * 128, 256)],
                    bufs[b].at[pl.ds(tr * 8, 8)],
                    ssems[b],
                )

        def stage_wait(tc, b):
            for tr in range(8):
                pltpu.make_async_copy(
                    wt_hbm.at[pl.ds(tr * 8, 8), pl.ds(tc * 128, 256)],
                    bufs[b].at[pl.ds(tr * 8, 8)],
                    ssems[b],
                ).wait()

        def shuffle(b):
            @plsc.parallel_loop(0, 16, unroll=4)
            def sbody(s):
                rot = jnp.bitwise_and(lanes + s, 15)
                dbase = lanes64 + rot
                for f0 in range(0, 64, 16):
                    for j0 in range(0, 256, 16):
                        vec = plsc.load_gather(
                            bufs[b], [f0 + rot, j0 + lanes]
                        )
                        plsc.store_scatter(
                            tbufs[b], [dbase + (j0 * 64 + f0)], vec
                        )

        def write_start(tc, b):
            pltpu.async_copy(
                tbufs[b], inter_hbm.at[pl.ds(tc * 8192, 16384)], wsems[b]
            )

        def write_wait(tc, b):
            pltpu.make_async_copy(
                tbufs[b], inter_hbm.at[pl.ds(tc * 8192, 16384)], wsems[b]
            ).wait()

        stage_start(start, 0)

        def body(r, carry):
            tc = start + 2 * r

            def one(b):
                stage_wait(tc, b)

                @pl.when(r + 1 < rounds)
                def _():
                    stage_start(tc + 2, 1 - b)

                @pl.when(r >= 2)
                def _():
                    write_wait(tc - 4, b)

                shuffle(b)
                write_start(tc, b)

            b = jnp.remainder(r, 2)

            @pl.when(b == 0)
            def _():
                one(0)

            @pl.when(b == 1)
            def _():
                one(1)

            return carry

        lax.fori_loop(0, rounds, body, 0)

        @pl.when(jnp.remainder(rounds, 2) == 0)
        def _():
            write_wait(start + 2 * rounds - 4, 0)
            write_wait(start + 2 * rounds - 2, 1)

        @pl.when(jnp.remainder(rounds, 2) == 1)
        def _():
            write_wait(start + 2 * rounds - 4, 1)
            write_wait(start + 2 * rounds - 2, 0)

        @pl.when(wid == _NW - 1)
        def _():
            pltpu.sync_copy(tail_hbm, tbuf0.at[pl.ds(0, _TAIL * _D)])
            pltpu.sync_copy(
                tbuf0.at[pl.ds(0, _TAIL * _D)],
                inter_hbm.at[pl.ds(_FULL_BLOCKS * 128 * _D, _TAIL * _D)],
            )

    return tcall(wt, tail1d)


def _gather_call(idxf, inter):
    nch, ch = idxf.shape
    cpw = nch // _NW
    mesh = plsc.VectorSubcoreMesh(core_axis_name="c", subcore_axis_name="s")

    @functools.partial(
        pl.kernel,
        out_type=jax.ShapeDtypeStruct((nch * ch * _D,), jnp.float32),
        mesh=mesh,
        scratch_types=[
            pltpu.VMEM((cpw, ch), jnp.int32),
            pltpu.VMEM((2 * ch, _D), jnp.float32),
            pltpu.VMEM((2 * ch, _D), jnp.float32),
            pltpu.VMEM((16384,), jnp.float32),
            pltpu.VMEM((16384,), jnp.float32),
            pltpu.SemaphoreType.DMA,
            pltpu.SemaphoreType.DMA,
            pltpu.SemaphoreType.DMA,
            pltpu.SemaphoreType.DMA,
        ],
        compiler_params=pltpu.CompilerParams(
            use_tc_tiling_on_sc=False, needs_layout_passes=False
        ),
    )
    def gcall(idx_hbm, inter_hbm, out_hbm, idx_v, buf0, buf1, tbuf0, tbuf1,
              gs0, gs1, ws0, ws1):
        bufs = (buf0, buf1)
        tbufs = (tbuf0, tbuf1)
        gsems = (gs0, gs1)
        wsems = (ws0, ws1)
        wid = lax.axis_index("s") * 2 + lax.axis_index("c")
        c0 = wid * cpw
        rounds = cpw // 2
        pltpu.sync_copy(idx_hbm.at[pl.ds(c0, cpw)], idx_v)
        lanes = lax.iota(jnp.int32, 16)
        dlanes = (lanes // 8) * 2048 + jnp.remainder(lanes, 8) * 128

        def g_start(r, b):
            pltpu.async_copy(
                inter_hbm.at[idx_v.at[2 * r]],
                bufs[b].at[pl.ds(0, ch)],
                gsems[b],
            )
            pltpu.async_copy(
                inter_hbm.at[idx_v.at[2 * r + 1]],
                bufs[b].at[pl.ds(ch, ch)],
                gsems[b],
            )

        def g_wait(r, b):
            for half in range(2):
                pltpu.make_async_copy(
                    inter_hbm.at[idx_v.at[2 * r + half]],
                    bufs[b].at[pl.ds(half * ch, ch)],
                    gsems[b],
                ).wait()

        def out_off(r, tr):
            c = c0 + 2 * r
            f = c // 128
            tc = c - f * 128
            return f * 1048576 + tr * 131072 + tc * 1024

        def write_start(r, b):
            for tr in range(8):
                pltpu.async_copy(
                    tbufs[b].at[pl.ds(tr * 2048, 2048)],
                    out_hbm.at[pl.ds(out_off(r, tr), 2048)],
                    wsems[b],
                )

        def write_wait(r, b):
            for tr in range(8):
                pltpu.make_async_copy(
                    tbufs[b].at[pl.ds(tr * 2048, 2048)],
                    out_hbm.at[pl.ds(out_off(r, tr), 2048)],
                    wsems[b],
                ).wait()

        def shuffle(b):
            @plsc.parallel_loop(0, 16, unroll=4)
            def sbody(s):
                rot = jnp.bitwise_and(lanes + s, 15)
                dbase = dlanes + rot
                for jj0 in range(0, 256, 16):
                    for d0 in range(0, 64, 16):
                        vec = plsc.load_gather(
                            bufs[b], [jj0 + rot, d0 + lanes]
                        )
                        const = (d0 // 8) * 2048 + (jj0 // 128) * 1024 + (
                            jj0 % 128
                        )
                        plsc.store_scatter(tbufs[b], [dbase + const], vec)

        g_start(0, 0)
        g_start(1, 1)

        def body(i, carry):
            for b in range(2):
                r = 2 * i + b
                g_wait(r, b)

                @pl.when(i >= 1)
                def _():
                    write_wait(r - 2, b)

                shuffle(b)
                write_start(r, b)
                g_start(r + 2, b)
            return carry

        lax.fori_loop(0, rounds // 2 - 1, body, 0)
        for b in range(2):
            r = rounds - 2 + b
            g_wait(r, b)
            write_wait(r - 2, b)
            shuffle(b)
            write_start(r, b)
            write_wait(r, b)

    return gcall(idxf, inter)


@jax.jit
def _embed(x, weight):
    batch, fields = x.shape
    idxf = x.T.reshape(fields * (batch // 128), 128).astype(jnp.int32)
    tail1d = weight[_FULL_BLOCKS * 128 :].reshape(_TAIL * _D)
    inter = _transpose_call(weight.T, tail1d).reshape(_V, _D)
    out1d = _gather_call(idxf, inter)
    out5 = out1d.reshape(fields, 8, batch // 128, 8, 128)
    return out5.transpose(2, 4, 0, 1, 3).reshape(batch, fields, _D)


def kernel(x, weight):
    return _embed(x, weight)
